# Initial kernel scaffold; baseline (speedup 1.0000x reference)
import jax
import jax.numpy as jnp
from jax.experimental import pallas as pl


def kernel(idx, targets, token_emb):
    raise NotImplementedError("write your pallas kernel here")


# speedup vs baseline: 1.6065x; 1.6065x over previous
import functools

import jax
import jax.numpy as jnp
from jax import lax
from jax.experimental import pallas as pl
from jax.experimental.pallas import tpu as pltpu
from jax.experimental.pallas import tpu_sc as plsc

VOCAB = 1000
N_TOK = 51200
NC, NS = 2, 16
NW = NC * NS
ROWS_PER_W = N_TOK // NW
CHUNK = 32
N_CHUNKS = ROWS_PER_W // CHUNK
LSE_PAD = 1024


def _lse_body(x_ref, o_ref):
    x = x_ref[...]
    m = jnp.max(x, axis=1)
    s = jnp.sum(jnp.exp(x - m[:, None]), axis=1)
    lse = m + jnp.log(s)
    o_ref[...] = jnp.concatenate(
        [lse, jnp.zeros((LSE_PAD - VOCAB,), jnp.float32)]
    )[:, None]


@jax.jit
def _lse_call(table):
    return pl.pallas_call(
        _lse_body,
        out_shape=jax.ShapeDtypeStruct((LSE_PAD, 1), jnp.float32),
    )(table)


def _sc_body(table, idxr, tr, lse, out, partials, idx_v, t_v, lse_v, buf,
             acc, semg):
    c_id = lax.axis_index("c")
    s_id = lax.axis_index("s")
    wid = s_id * NC + c_id
    pltpu.sync_copy(idxr.at[wid], idx_v)
    pltpu.sync_copy(tr.at[wid], t_v)
    pltpu.sync_copy(lse, lse_v)
    acc[...] = jnp.zeros((16,), jnp.float32)
    base = wid * ROWS_PER_W

    def chunk_step(c, carry):
        pltpu.async_copy(table.at[idx_v.at[c]], buf, semg).wait()
        for g in range(CHUNK // 16):
            rows = lax.iota(jnp.int32, 16) + g * 16
            tv = t_v[c, pl.ds(g * 16, 16)]
            iv = idx_v[c, pl.ds(g * 16, 16)]
            vals = plsc.load_gather(buf, [rows, tv])
            lsev = plsc.load_gather(lse_v, [iv])
            acc[...] = acc[...] + (lsev - vals)
        pltpu.sync_copy(buf, out.at[pl.ds(base + c * CHUNK, CHUNK)])
        return carry

    lax.fori_loop(0, N_CHUNKS, chunk_step, 0)
    pltpu.sync_copy(acc, partials.at[wid])


@jax.jit
def _sc_call(table, idx_r, t_r, lse_flat):
    mesh = plsc.VectorSubcoreMesh(
        core_axis_name="c", subcore_axis_name="s", num_cores=NC,
        num_subcores=NS,
    )
    return pl.kernel(
        _sc_body,
        out_type=(
            jax.ShapeDtypeStruct((N_TOK, VOCAB), jnp.float32),
            jax.ShapeDtypeStruct((NW, 16), jnp.float32),
        ),
        mesh=mesh,
        compiler_params=pltpu.CompilerParams(
            use_tc_tiling_on_sc=False, needs_layout_passes=False
        ),
        scratch_types=[
            pltpu.VMEM((N_CHUNKS, CHUNK), jnp.int32),
            pltpu.VMEM((N_CHUNKS, CHUNK), jnp.int32),
            pltpu.VMEM((LSE_PAD,), jnp.float32),
            pltpu.VMEM((CHUNK, VOCAB), jnp.float32),
            pltpu.VMEM((16,), jnp.float32),
            pltpu.SemaphoreType.DMA,
        ],
    )(table, idx_r, t_r, lse_flat)


def _loss_body(p_ref, o_ref):
    o_ref[...] = (jnp.sum(p_ref[...]) / N_TOK).reshape(1, 1)


@jax.jit
def _loss_call(partials):
    return pl.pallas_call(
        _loss_body,
        out_shape=jax.ShapeDtypeStruct((1, 1), jnp.float32),
    )(partials)


def kernel(idx, targets, token_emb):
    idx_r = idx.reshape(NW, N_CHUNKS, CHUNK).astype(jnp.int32)
    t_r = targets.reshape(NW, N_CHUNKS, CHUNK).astype(jnp.int32)
    lse = _lse_call(token_emb).reshape(LSE_PAD)
    logits2, partials = _sc_call(token_emb, idx_r, t_r, lse)
    loss = _loss_call(partials)[0, 0]
    return logits2, loss


# speedup vs baseline: 1.6879x; 1.0507x over previous
import functools

import jax
import jax.numpy as jnp
from jax import lax
from jax.experimental import pallas as pl
from jax.experimental.pallas import tpu as pltpu
from jax.experimental.pallas import tpu_sc as plsc

VOCAB = 1000
N_TOK = 51200
NC, NS = 2, 16
NW = NC * NS
ROWS_PER_W = N_TOK // NW
CHUNK = 32
N_CHUNKS = ROWS_PER_W // CHUNK
LSE_PAD = 1024


def _lse_body(x_ref, o_ref):
    x = x_ref[...]
    m = jnp.max(x, axis=1)
    s = jnp.sum(jnp.exp(x - m[:, None]), axis=1)
    lse = m + jnp.log(s)
    o_ref[...] = jnp.concatenate(
        [lse, jnp.zeros((LSE_PAD - VOCAB,), jnp.float32)]
    )[:, None]


@jax.jit
def _lse_call(table):
    return pl.pallas_call(
        _lse_body,
        out_shape=jax.ShapeDtypeStruct((LSE_PAD, 1), jnp.float32),
    )(table)


def _sc_body(table, idxr, tr, lse, out, partials, idx_v, t_v, lse_v, buf,
             acc, semg, sems):
    c_id = lax.axis_index("c")
    s_id = lax.axis_index("s")
    wid = s_id * NC + c_id
    pltpu.sync_copy(idxr.at[wid], idx_v)
    pltpu.sync_copy(tr.at[wid], t_v)
    pltpu.sync_copy(lse, lse_v)
    acc[...] = jnp.zeros((16,), jnp.float32)
    base = wid * ROWS_PER_W

    def gather_desc(c, b):
        return pltpu.make_async_copy(
            table.at[idx_v.at[c]], buf.at[b], semg.at[b]
        )

    def scatter_desc(c, b):
        return pltpu.make_async_copy(
            buf.at[b], out.at[pl.ds(base + c * CHUNK, CHUNK)], sems.at[b]
        )

    gather_desc(0, 0).start()

    def step(k, carry):
        for b in range(2):
            c = 2 * k + b
            ob = 1 - b
            gather_desc(c, b).wait()

            @pl.when(c + 1 < N_CHUNKS)
            def _start_next():
                @pl.when(c >= 1)
                def _drain():
                    scatter_desc(c - 1, ob).wait()

                gather_desc(c + 1, ob).start()

            for g in range(CHUNK // 16):
                rows = lax.iota(jnp.int32, 16) + g * 16
                tv = t_v[c, pl.ds(g * 16, 16)]
                iv = idx_v[c, pl.ds(g * 16, 16)]
                vals = plsc.load_gather(buf.at[b], [rows, tv])
                lsev = plsc.load_gather(lse_v, [iv])
                acc[...] = acc[...] + (lsev - vals)
            scatter_desc(c, b).start()
        return carry

    lax.fori_loop(0, N_CHUNKS // 2, step, 0)
    scatter_desc(N_CHUNKS - 2, 0).wait()
    scatter_desc(N_CHUNKS - 1, 1).wait()
    pltpu.sync_copy(acc, partials.at[wid])


@jax.jit
def _sc_call(table, idx_r, t_r, lse_flat):
    mesh = plsc.VectorSubcoreMesh(
        core_axis_name="c", subcore_axis_name="s", num_cores=NC,
        num_subcores=NS,
    )
    return pl.kernel(
        _sc_body,
        out_type=(
            jax.ShapeDtypeStruct((N_TOK, VOCAB), jnp.float32),
            jax.ShapeDtypeStruct((NW, 16), jnp.float32),
        ),
        mesh=mesh,
        compiler_params=pltpu.CompilerParams(
            use_tc_tiling_on_sc=False, needs_layout_passes=False
        ),
        scratch_types=[
            pltpu.VMEM((N_CHUNKS, CHUNK), jnp.int32),
            pltpu.VMEM((N_CHUNKS, CHUNK), jnp.int32),
            pltpu.VMEM((LSE_PAD,), jnp.float32),
            pltpu.VMEM((2, CHUNK, VOCAB), jnp.float32),
            pltpu.VMEM((16,), jnp.float32),
            pltpu.SemaphoreType.DMA((2,)),
            pltpu.SemaphoreType.DMA((2,)),
        ],
    )(table, idx_r, t_r, lse_flat)


def _loss_body(p_ref, o_ref):
    o_ref[...] = (jnp.sum(p_ref[...]) / N_TOK).reshape(1, 1)


@jax.jit
def _loss_call(partials):
    return pl.pallas_call(
        _loss_body,
        out_shape=jax.ShapeDtypeStruct((1, 1), jnp.float32),
    )(partials)


def kernel(idx, targets, token_emb):
    idx_r = idx.reshape(NW, N_CHUNKS, CHUNK).astype(jnp.int32)
    t_r = targets.reshape(NW, N_CHUNKS, CHUNK).astype(jnp.int32)
    lse = _lse_call(token_emb).reshape(LSE_PAD)
    logits2, partials = _sc_call(token_emb, idx_r, t_r, lse)
    loss = _loss_call(partials)[0, 0]
    return logits2, loss
